# baseline (speedup 1.0000x reference)
import functools

import jax
import jax.numpy as jnp
from jax import lax
from jax.experimental import pallas as pl
from jax.experimental.pallas import tpu as pltpu
from jax.experimental.pallas import tpu_sc as plsc

B = 16384
C = 1000
NC = 2
NS = 16
L = 16
NW = NC * NS
RPT = B // NW
CHUNKS = RPT // L
IW = 128
IROWS = RPT // IW


@functools.partial(
    pl.kernel,
    mesh=plsc.VectorSubcoreMesh(core_axis_name="c", subcore_axis_name="s"),
    out_type=jax.ShapeDtypeStruct((NW, L), jnp.float32),
    scratch_types=[
        pltpu.VMEM((RPT,), jnp.int32),
        pltpu.VMEM((RPT,), jnp.int32),
        pltpu.VMEM((RPT,), jnp.float32),
        pltpu.VMEM((IROWS, IW), jnp.int32),
        pltpu.VMEM((IROWS, IW), jnp.int32),
        pltpu.VMEM((IROWS, IW), jnp.float32),
        pltpu.VMEM((IROWS, IW), jnp.float32),
        pltpu.VMEM((L,), jnp.float32),
        pltpu.SemaphoreType.DMA,
    ],
)
def _sc_partials(z_hbm, l_hbm, lp_hbm, cond_hbm, out_hbm,
                 l_v, lp_v, cond_v, idxp, idxn, pos, neg, acc_v, sem):
    cidx = lax.axis_index("c")
    sidx = lax.axis_index("s")
    wid = cidx * NS + sidx
    base = wid * RPT

    pltpu.sync_copy(l_hbm.at[pl.ds(base, RPT)], l_v)
    pltpu.sync_copy(lp_hbm.at[pl.ds(base, RPT)], lp_v)
    pltpu.sync_copy(cond_hbm.at[pl.ds(base, RPT)], cond_v)

    lane = lax.iota(jnp.int32, L)
    for t in range(CHUNKS):
        rows = (base + t * L) * C + lane * C
        r, c0 = t // (IW // L), (t % (IW // L)) * L
        idxp[r, pl.ds(c0, L)] = rows + l_v[pl.ds(t * L, L)]
        idxn[r, pl.ds(c0, L)] = rows + lp_v[pl.ds(t * L, L)]

    copies = []
    for r in range(IROWS):
        copies.append(pltpu.async_copy(z_hbm.at[idxp.at[r]], pos.at[r], sem))
        copies.append(pltpu.async_copy(z_hbm.at[idxn.at[r]], neg.at[r], sem))
    for cp in copies:
        cp.wait()

    acc = jnp.zeros((L,), jnp.float32)
    for t in range(CHUNKS):
        r, c0 = t // (IW // L), (t % (IW // L)) * L
        acc = acc + (pos[r, pl.ds(c0, L)] - neg[r, pl.ds(c0, L)]) \
            * cond_v[pl.ds(t * L, L)]
    acc_v[...] = acc
    pltpu.sync_copy(acc_v, out_hbm.at[wid])


@functools.partial(
    pl.kernel,
    mesh=plsc.VectorSubcoreMesh(core_axis_name="c", subcore_axis_name="s",
                                num_cores=1, num_subcores=1),
    out_type=jax.ShapeDtypeStruct((L,), jnp.float32),
    scratch_types=[
        pltpu.VMEM((NW, L), jnp.float32),
        pltpu.VMEM((L,), jnp.float32),
    ],
)
def _sc_finalize(part_hbm, out_hbm, psum_v, out_v):
    pltpu.sync_copy(part_hbm, psum_v)
    tot = jnp.zeros((L,), jnp.float32)
    for i in range(NW):
        tot = tot + psum_v[i]
    tv = jnp.zeros((L,), jnp.float32)
    for j in range(L):
        tv = tv + jnp.full((L,), tot[j], jnp.float32)
    out_v[...] = tv
    pltpu.sync_copy(out_v, out_hbm)


def kernel(z, condition, l, l_prime):
    zf = z.reshape(-1)
    li = l.astype(jnp.int32)
    lpi = l_prime.astype(jnp.int32)
    condf = condition.astype(jnp.float32)
    partials = _sc_partials(zf, li, lpi, condf)
    return _sc_finalize(partials)[0]


# speedup vs baseline: 1.2767x; 1.2767x over previous
import functools

import jax
import jax.numpy as jnp
from jax import lax
from jax.experimental import pallas as pl
from jax.experimental.pallas import tpu as pltpu
from jax.experimental.pallas import tpu_sc as plsc

B = 16384
C = 1000
NC = 2
NS = 16
L = 16
NW = NC * NS
RPT = B // NW
GRP = 16
NGRP = RPT // GRP
SCP = 5
SPLIT_COL = SCP * 128

TC_BC = 128
TC_J0 = SPLIT_COL // TC_BC
TC_NJ = 8 - TC_J0
TC_BR = 1024
TC_NI = B // TC_BR


@functools.partial(
    pl.kernel,
    mesh=plsc.VectorSubcoreMesh(core_axis_name="c", subcore_axis_name="s"),
    out_type=jax.ShapeDtypeStruct((NW, L), jnp.float32),
    scratch_types=[
        pltpu.VMEM((RPT,), jnp.int32),
        pltpu.VMEM((RPT,), jnp.int32),
        pltpu.VMEM((RPT,), jnp.float32),
        pltpu.VMEM((2 * 2 * SCP * 8, 128), jnp.float32),
        pltpu.VMEM((L,), jnp.float32),
        pltpu.SemaphoreType.DMA,
        pltpu.SemaphoreType.DMA,
    ],
    compiler_params=pltpu.CompilerParams(use_tc_tiling_on_sc=True,
                                         needs_layout_passes=False),
)
def _sc_stream(z_hbm, l_hbm, lp_hbm, cond_hbm, out_hbm,
               l_v, lp_v, cond_v, buf, acc_v, sem0, sem1):
    cidx = lax.axis_index("c")
    sidx = lax.axis_index("s")
    wid = cidx * NS + sidx
    base = wid * RPT

    pltpu.sync_copy(l_hbm.at[pl.ds(base, RPT)], l_v)
    pltpu.sync_copy(lp_hbm.at[pl.ds(base, RPT)], lp_v)
    pltpu.sync_copy(cond_hbm.at[pl.ds(base, RPT)], cond_v)

    lane = lax.iota(jnp.int32, L)
    grp_ix = lane // 8
    row_ix = lane % 8

    def issue(g, slot, sem):
        row0 = base + g * GRP
        for gi in range(2):
            r = pl.multiple_of(row0 + 8 * gi, 8)
            for p in range(SCP):
                pltpu.async_copy(
                    z_hbm.at[pl.ds(r, 8), pl.ds(p * 128, 128)],
                    buf.at[pl.ds(((slot * 2 + gi) * SCP + p) * 8, 8)], sem)

    def drain(slot, sem):
        for _ in range(2 * SCP):
            pltpu.make_async_copy(
                z_hbm.at[pl.ds(0, 8), pl.ds(0, 128)],
                buf.at[pl.ds(slot * 2 * SCP * 8, 8)], sem).wait()

    def process(g, slot, acc):
        off = g * GRP
        slot_v = jnp.full((L,), slot, jnp.int32)
        lg = l_v[pl.ds(off, GRP)]
        lpg = lp_v[pl.ds(off, GRP)]
        cg = cond_v[pl.ds(off, GRP)]
        mp = lg < SPLIT_COL
        mn = lpg < SPLIT_COL
        pgp = jnp.minimum(lg // 128, SCP - 1)
        pgn = jnp.minimum(lpg // 128, SCP - 1)
        rowp = ((slot_v * 2 + grp_ix) * SCP + pgp) * 8 + row_ix
        rown = ((slot_v * 2 + grp_ix) * SCP + pgn) * 8 + row_ix
        pos = plsc.load_gather(buf, [rowp, lg % 128], mask=mp)
        neg = plsc.load_gather(buf, [rown, lpg % 128], mask=mn)
        pos = jnp.where(mp, pos, 0.0)
        neg = jnp.where(mn, neg, 0.0)
        return acc + (pos - neg) * cg

    issue(0, 0, sem0)

    def body(k, acc):
        g0 = 2 * k
        issue(g0 + 1, 1, sem1)
        drain(0, sem0)
        acc = process(g0, 0, acc)

        @pl.when(g0 + 2 < NGRP)
        def _():
            issue(g0 + 2, 0, sem0)
        drain(1, sem1)
        acc = process(g0 + 1, 1, acc)
        return acc

    acc = lax.fori_loop(0, NGRP // 2, body, jnp.zeros((L,), jnp.float32))
    acc_v[...] = acc
    pltpu.sync_copy(acc_v, out_hbm.at[wid])


def _tc_dense_body(z_ref, l_ref, lp_ref, cond_ref, out_ref):
    i = pl.program_id(0)
    j = pl.program_id(1)

    @pl.when((i == 0) & (j == 0))
    def _():
        out_ref[...] = jnp.zeros((1, 1), jnp.float32)

    zb = z_ref[...]
    lb = l_ref[0, 0, :].reshape(TC_BR, 1)
    lpb = lp_ref[0, 0, :].reshape(TC_BR, 1)
    cb = cond_ref[0, 0, :].reshape(TC_BR, 1)
    col0 = (TC_J0 + j) * TC_BC
    cols = jax.lax.broadcasted_iota(jnp.int32, (TC_BR, TC_BC), 1) + col0
    val = jnp.where(cols == lb, zb, 0.0) - jnp.where(cols == lpb, zb, 0.0)
    out_ref[...] += jnp.sum(val * cb).reshape(1, 1)


_tc_dense = pl.pallas_call(
    _tc_dense_body,
    grid=(TC_NI, TC_NJ),
    in_specs=[
        pl.BlockSpec((TC_BR, TC_BC), lambda i, j: (i, TC_J0 + j)),
        pl.BlockSpec((1, 1, TC_BR), lambda i, j: (i, 0, 0)),
        pl.BlockSpec((1, 1, TC_BR), lambda i, j: (i, 0, 0)),
        pl.BlockSpec((1, 1, TC_BR), lambda i, j: (i, 0, 0)),
    ],
    out_specs=pl.BlockSpec((1, 1), lambda i, j: (0, 0)),
    out_shape=jax.ShapeDtypeStruct((1, 1), jnp.float32),
    compiler_params=pltpu.CompilerParams(
        dimension_semantics=("arbitrary", "arbitrary")),
)


def _tc_combine_body(part_ref, tcs_ref, out_ref):
    out_ref[...] = (jnp.sum(part_ref[...]) + tcs_ref[0, 0]).reshape(1, 1)


_tc_combine = pl.pallas_call(
    _tc_combine_body,
    out_shape=jax.ShapeDtypeStruct((1, 1), jnp.float32),
)


def kernel(z, condition, l, l_prime):
    li = l.astype(jnp.int32)
    lpi = l_prime.astype(jnp.int32)
    condf = condition.astype(jnp.float32)
    l3 = li.reshape(TC_NI, 1, TC_BR)
    lp3 = lpi.reshape(TC_NI, 1, TC_BR)
    c3 = condf.reshape(TC_NI, 1, TC_BR)
    sc_part = _sc_stream(z, li, lpi, condf)
    tc_part = _tc_dense(z, l3, lp3, c3)
    return _tc_combine(sc_part, tc_part)[0, 0]


# speedup vs baseline: 1.9146x; 1.4996x over previous
import functools

import jax
import jax.numpy as jnp
from jax import lax
from jax.experimental import pallas as pl
from jax.experimental.pallas import tpu as pltpu
from jax.experimental.pallas import tpu_sc as plsc

B = 16384
C = 1000
NC = 2
NS = 16
L = 16
NW = NC * NS
RPT = B // NW
CS = 64
SPLIT_COL = 384
NSTG = SPLIT_COL // CS
ZREDIR = 2 * CS

TC_BC = C - SPLIT_COL
TC_BI = 2048
TC_NI = B // TC_BI


@functools.partial(
    pl.kernel,
    mesh=plsc.VectorSubcoreMesh(core_axis_name="c", subcore_axis_name="s"),
    out_type=jax.ShapeDtypeStruct((NW, L), jnp.float32),
    scratch_types=[
        pltpu.VMEM((RPT,), jnp.int32),
        pltpu.VMEM((RPT,), jnp.int32),
        pltpu.VMEM((RPT,), jnp.int32),
        pltpu.VMEM((ZREDIR + 8, RPT), jnp.float32),
        pltpu.VMEM((L,), jnp.float32),
        pltpu.SemaphoreType.DMA,
        pltpu.SemaphoreType.DMA,
    ],
)
def _sc_stream(zt_hbm, l_hbm, lp_hbm, cond_hbm, out_hbm,
               l_v, lp_v, cond_v, buf, acc_v, sem0, sem1):
    cidx = lax.axis_index("c")
    sidx = lax.axis_index("s")
    wid = cidx * NS + sidx
    base = wid * RPT

    pltpu.sync_copy(l_hbm.at[pl.ds(base, RPT)], l_v)
    pltpu.sync_copy(lp_hbm.at[pl.ds(base, RPT)], lp_v)
    pltpu.sync_copy(cond_hbm.at[pl.ds(base, RPT)], cond_v)

    zv = jnp.zeros((L,), jnp.float32)
    for k in range(RPT // L):
        buf[ZREDIR, pl.ds(k * L, L)] = zv

    lane = lax.iota(jnp.int32, L)

    def issue(s, slot, sem):
        pltpu.async_copy(
            zt_hbm.at[pl.ds(s * CS, CS), pl.ds(base, RPT)],
            buf.at[pl.ds(slot * CS, CS)], sem)

    def drain(slot, sem):
        pltpu.make_async_copy(
            zt_hbm.at[pl.ds(0, CS), pl.ds(0, RPT)],
            buf.at[pl.ds(slot * CS, CS)], sem).wait()

    def process(s, slot, acc):
        c_lo = s * CS

        def inner(g, acc):
            off = g * L
            lg = l_v[pl.ds(off, L)]
            lpg = lp_v[pl.ds(off, L)]
            cndg = cond_v[pl.ds(off, L)]
            for t in range(L):
                l_r = lg[t]
                lp_r = lpg[t]
                c_r = cndg[t]
                okp = (c_r != 0) & (l_r >= c_lo) & (l_r < c_lo + CS)
                okn = (c_r != 0) & (lp_r >= c_lo) & (lp_r < c_lo + CS)
                rowp = jnp.where(okp, slot * CS + l_r - c_lo, ZREDIR)
                rown = jnp.where(okn, slot * CS + lp_r - c_lo, ZREDIR)
                chp = buf[rowp, pl.ds(off, L)]
                chn = buf[rown, pl.ds(off, L)]
                sel = lane == t
                acc = acc + jnp.where(sel, chp, 0.0) \
                    - jnp.where(sel, chn, 0.0)
            return acc

        return lax.fori_loop(0, RPT // L, inner, acc)

    issue(0, 0, sem0)

    def body(k, acc):
        s0 = 2 * k
        issue(s0 + 1, 1, sem1)
        drain(0, sem0)
        acc = process(s0, 0, acc)

        @pl.when(s0 + 2 < NSTG)
        def _():
            issue(s0 + 2, 0, sem0)
        drain(1, sem1)
        acc = process(s0 + 1, 1, acc)
        return acc

    acc = lax.fori_loop(0, NSTG // 2, body, jnp.zeros((L,), jnp.float32))
    acc_v[...] = acc
    pltpu.sync_copy(acc_v, out_hbm.at[wid])


def _tc_block_copy(zt_any, vbuf, sems, i, slot):
    return pltpu.make_async_copy(
        zt_any.at[pl.ds(SPLIT_COL, TC_BC), pl.ds(i * TC_BI, TC_BI)],
        vbuf.at[slot], sems.at[slot])


def _tc_dense_body(zt_any, l_ref, lp_ref, cond_ref, out_ref,
                   vbuf, acc_ref, sems):
    i = pl.program_id(0)
    slot = lax.rem(i, 2)

    @pl.when(i == 0)
    def _():
        acc_ref[...] = jnp.zeros((1, TC_BI), jnp.float32)
        _tc_block_copy(zt_any, vbuf, sems, 0, 0).start()

    @pl.when(i + 1 < TC_NI)
    def _():
        _tc_block_copy(zt_any, vbuf, sems, i + 1, lax.rem(i + 1, 2)).start()

    _tc_block_copy(zt_any, vbuf, sems, i, slot).wait()

    zb = vbuf[slot]
    lb = l_ref[0, 0, :].reshape(1, TC_BI)
    lpb = lp_ref[0, 0, :].reshape(1, TC_BI)
    cb = cond_ref[0, 0, :].reshape(1, TC_BI)
    cids = jax.lax.broadcasted_iota(
        jnp.int32, (TC_BC, TC_BI), 0) + SPLIT_COL
    val = jnp.where(cids == lb, zb, 0.0) - jnp.where(cids == lpb, zb, 0.0)
    acc_ref[...] += jnp.sum(val, axis=0, keepdims=True) * cb

    @pl.when(i == TC_NI - 1)
    def _():
        out_ref[...] = jnp.sum(acc_ref[...]).reshape(1, 1)


_tc_dense = pl.pallas_call(
    _tc_dense_body,
    grid=(TC_NI,),
    in_specs=[
        pl.BlockSpec(memory_space=pl.ANY),
        pl.BlockSpec((1, 1, TC_BI), lambda i: (i, 0, 0)),
        pl.BlockSpec((1, 1, TC_BI), lambda i: (i, 0, 0)),
        pl.BlockSpec((1, 1, TC_BI), lambda i: (i, 0, 0)),
    ],
    out_specs=pl.BlockSpec((1, 1), lambda i: (0, 0)),
    out_shape=jax.ShapeDtypeStruct((1, 1), jnp.float32),
    scratch_shapes=[pltpu.VMEM((2, TC_BC, TC_BI), jnp.float32),
                    pltpu.VMEM((1, TC_BI), jnp.float32),
                    pltpu.SemaphoreType.DMA((2,))],
    compiler_params=pltpu.CompilerParams(
        dimension_semantics=("arbitrary",)),
)


def _tc_combine_body(part_ref, tcs_ref, out_ref):
    out_ref[...] = (jnp.sum(part_ref[...]) + tcs_ref[0, 0]).reshape(1, 1)


_tc_combine = pl.pallas_call(
    _tc_combine_body,
    out_shape=jax.ShapeDtypeStruct((1, 1), jnp.float32),
)


def kernel(z, condition, l, l_prime):
    zt = jnp.swapaxes(z, 0, 1)
    li = l.astype(jnp.int32)
    lpi = l_prime.astype(jnp.int32)
    condi = condition.astype(jnp.int32)
    condf = condition.astype(jnp.float32)
    l3 = li.reshape(TC_NI, 1, TC_BI)
    lp3 = lpi.reshape(TC_NI, 1, TC_BI)
    c3 = condf.reshape(TC_NI, 1, TC_BI)
    sc_part = _sc_stream(zt, li, lpi, condi)
    tc_part = _tc_dense(zt, l3, lp3, c3)
    return _tc_combine(sc_part, tc_part)[0, 0]


# speedup vs baseline: 3.1782x; 1.6600x over previous
import functools

import jax
import jax.numpy as jnp
from jax import lax
from jax.experimental import pallas as pl
from jax.experimental.pallas import tpu as pltpu
from jax.experimental.pallas import tpu_sc as plsc

B = 16384
C = 1000
NC = 2
NS = 16
L = 16
NW = NC * NS
RPT = B // NW
CS = 64
SPLIT_COL = 128
NSTG = SPLIT_COL // CS
ZREDIR = 2 * CS

TC_BC = C - SPLIT_COL
TC_BI = 4096
TC_NI = B // TC_BI


@functools.partial(
    pl.kernel,
    mesh=plsc.VectorSubcoreMesh(core_axis_name="c", subcore_axis_name="s"),
    out_type=jax.ShapeDtypeStruct((NW, L), jnp.float32),
    scratch_types=[
        pltpu.VMEM((RPT,), jnp.int32),
        pltpu.VMEM((RPT,), jnp.int32),
        pltpu.VMEM((RPT,), jnp.int32),
        pltpu.VMEM((ZREDIR + 8, RPT), jnp.float32),
        pltpu.VMEM((L,), jnp.float32),
        pltpu.SemaphoreType.DMA,
        pltpu.SemaphoreType.DMA,
    ],
)
def _sc_stream(zt_hbm, l_hbm, lp_hbm, cond_hbm, out_hbm,
               l_v, lp_v, cond_v, buf, acc_v, sem0, sem1):
    cidx = lax.axis_index("c")
    sidx = lax.axis_index("s")
    wid = cidx * NS + sidx
    base = wid * RPT

    pltpu.sync_copy(l_hbm.at[pl.ds(base, RPT)], l_v)
    pltpu.sync_copy(lp_hbm.at[pl.ds(base, RPT)], lp_v)
    pltpu.sync_copy(cond_hbm.at[pl.ds(base, RPT)], cond_v)

    zv = jnp.zeros((L,), jnp.float32)
    for k in range(RPT // L):
        buf[ZREDIR, pl.ds(k * L, L)] = zv

    lane = lax.iota(jnp.int32, L)

    def issue(s, slot, sem):
        pltpu.async_copy(
            zt_hbm.at[pl.ds(s * CS, CS), pl.ds(base, RPT)],
            buf.at[pl.ds(slot * CS, CS)], sem)

    def drain(slot, sem):
        pltpu.make_async_copy(
            zt_hbm.at[pl.ds(0, CS), pl.ds(0, RPT)],
            buf.at[pl.ds(slot * CS, CS)], sem).wait()

    def process(s, slot, acc):
        c_lo = s * CS

        def inner(g, acc):
            off = g * L
            lg = l_v[pl.ds(off, L)]
            lpg = lp_v[pl.ds(off, L)]
            cndg = cond_v[pl.ds(off, L)]
            for t in range(L):
                l_r = lg[t]
                lp_r = lpg[t]
                c_r = cndg[t]
                okp = (c_r != 0) & (l_r >= c_lo) & (l_r < c_lo + CS)
                okn = (c_r != 0) & (lp_r >= c_lo) & (lp_r < c_lo + CS)
                rowp = jnp.where(okp, slot * CS + l_r - c_lo, ZREDIR)
                rown = jnp.where(okn, slot * CS + lp_r - c_lo, ZREDIR)
                chp = buf[rowp, pl.ds(off, L)]
                chn = buf[rown, pl.ds(off, L)]
                sel = lane == t
                acc = acc + jnp.where(sel, chp, 0.0) \
                    - jnp.where(sel, chn, 0.0)
            return acc

        return lax.fori_loop(0, RPT // L, inner, acc)

    issue(0, 0, sem0)

    def body(k, acc):
        s0 = 2 * k
        issue(s0 + 1, 1, sem1)
        drain(0, sem0)
        acc = process(s0, 0, acc)

        @pl.when(s0 + 2 < NSTG)
        def _():
            issue(s0 + 2, 0, sem0)
        drain(1, sem1)
        acc = process(s0 + 1, 1, acc)
        return acc

    acc = lax.fori_loop(0, NSTG // 2, body, jnp.zeros((L,), jnp.float32))
    acc_v[...] = acc
    pltpu.sync_copy(acc_v, out_hbm.at[wid])


def _tc_block_copy(zt_any, vbuf, sems, i, slot):
    return pltpu.make_async_copy(
        zt_any.at[pl.ds(SPLIT_COL, TC_BC), pl.ds(i * TC_BI, TC_BI)],
        vbuf.at[slot], sems.at[slot])


def _tc_dense_body(zt_any, l_ref, lp_ref, cond_ref, out_ref,
                   vbuf, acc_ref, sems):
    i = pl.program_id(0)
    slot = lax.rem(i, 2)

    @pl.when(i == 0)
    def _():
        acc_ref[...] = jnp.zeros((1, TC_BI), jnp.float32)
        _tc_block_copy(zt_any, vbuf, sems, 0, 0).start()

    @pl.when(i + 1 < TC_NI)
    def _():
        _tc_block_copy(zt_any, vbuf, sems, i + 1, lax.rem(i + 1, 2)).start()

    _tc_block_copy(zt_any, vbuf, sems, i, slot).wait()

    zb = vbuf[slot]
    lb = l_ref[0, 0, :].reshape(1, TC_BI)
    lpb = lp_ref[0, 0, :].reshape(1, TC_BI)
    cb = cond_ref[0, 0, :].reshape(1, TC_BI)
    cids = jax.lax.broadcasted_iota(
        jnp.int32, (TC_BC, TC_BI), 0) + SPLIT_COL
    val = jnp.where(cids == lb, zb, 0.0) - jnp.where(cids == lpb, zb, 0.0)
    acc_ref[...] += jnp.sum(val, axis=0, keepdims=True) * cb

    @pl.when(i == TC_NI - 1)
    def _():
        out_ref[...] = jnp.sum(acc_ref[...]).reshape(1, 1)


_tc_dense = pl.pallas_call(
    _tc_dense_body,
    grid=(TC_NI,),
    in_specs=[
        pl.BlockSpec(memory_space=pl.ANY),
        pl.BlockSpec((1, 1, TC_BI), lambda i: (i, 0, 0)),
        pl.BlockSpec((1, 1, TC_BI), lambda i: (i, 0, 0)),
        pl.BlockSpec((1, 1, TC_BI), lambda i: (i, 0, 0)),
    ],
    out_specs=pl.BlockSpec((1, 1), lambda i: (0, 0)),
    out_shape=jax.ShapeDtypeStruct((1, 1), jnp.float32),
    scratch_shapes=[pltpu.VMEM((2, TC_BC, TC_BI), jnp.float32),
                    pltpu.VMEM((1, TC_BI), jnp.float32),
                    pltpu.SemaphoreType.DMA((2,))],
    compiler_params=pltpu.CompilerParams(
        dimension_semantics=("arbitrary",)),
)


def _tc_combine_body(part_ref, tcs_ref, out_ref):
    out_ref[...] = (jnp.sum(part_ref[...]) + tcs_ref[0, 0]).reshape(1, 1)


_tc_combine = pl.pallas_call(
    _tc_combine_body,
    out_shape=jax.ShapeDtypeStruct((1, 1), jnp.float32),
)


def kernel(z, condition, l, l_prime):
    zt = jnp.swapaxes(z, 0, 1)
    li = l.astype(jnp.int32)
    lpi = l_prime.astype(jnp.int32)
    condi = condition.astype(jnp.int32)
    condf = condition.astype(jnp.float32)
    l3 = li.reshape(TC_NI, 1, TC_BI)
    lp3 = lpi.reshape(TC_NI, 1, TC_BI)
    c3 = condf.reshape(TC_NI, 1, TC_BI)
    sc_part = _sc_stream(zt, li, lpi, condi)
    tc_part = _tc_dense(zt, l3, lp3, c3)
    return _tc_combine(sc_part, tc_part)[0, 0]


# speedup vs baseline: 3.4616x; 1.0892x over previous
import functools

import jax
import jax.numpy as jnp
from jax import lax
from jax.experimental import pallas as pl
from jax.experimental.pallas import tpu as pltpu
from jax.experimental.pallas import tpu_sc as plsc

B = 16384
C = 1000
NC = 2
NS = 16
L = 16
NW = NC * NS
RPT = B // NW
CS = 64
SPLIT_COL = 64
NSTG = SPLIT_COL // CS
ZREDIR = 2 * CS

TC_BC = C - SPLIT_COL
TC_BI = 4096
TC_NI = B // TC_BI


@functools.partial(
    pl.kernel,
    mesh=plsc.VectorSubcoreMesh(core_axis_name="c", subcore_axis_name="s"),
    out_type=jax.ShapeDtypeStruct((NW, L), jnp.float32),
    scratch_types=[
        pltpu.VMEM((RPT,), jnp.int32),
        pltpu.VMEM((RPT,), jnp.int32),
        pltpu.VMEM((RPT,), jnp.int32),
        pltpu.VMEM((ZREDIR + 8, RPT), jnp.float32),
        pltpu.VMEM((L,), jnp.float32),
        pltpu.SemaphoreType.DMA,
        pltpu.SemaphoreType.DMA,
    ],
)
def _sc_stream(zt_hbm, l_hbm, lp_hbm, cond_hbm, out_hbm,
               l_v, lp_v, cond_v, buf, acc_v, sem0, sem1):
    cidx = lax.axis_index("c")
    sidx = lax.axis_index("s")
    wid = cidx * NS + sidx
    base = wid * RPT

    pltpu.sync_copy(l_hbm.at[pl.ds(base, RPT)], l_v)
    pltpu.sync_copy(lp_hbm.at[pl.ds(base, RPT)], lp_v)
    pltpu.sync_copy(cond_hbm.at[pl.ds(base, RPT)], cond_v)

    zv = jnp.zeros((L,), jnp.float32)
    for k in range(RPT // L):
        buf[ZREDIR, pl.ds(k * L, L)] = zv

    lane = lax.iota(jnp.int32, L)

    def issue(s, slot, sem):
        pltpu.async_copy(
            zt_hbm.at[pl.ds(s * CS, CS), pl.ds(base, RPT)],
            buf.at[pl.ds(slot * CS, CS)], sem)

    def drain(slot, sem):
        pltpu.make_async_copy(
            zt_hbm.at[pl.ds(0, CS), pl.ds(0, RPT)],
            buf.at[pl.ds(slot * CS, CS)], sem).wait()

    def process(s, slot, acc):
        c_lo = s * CS

        def inner(g, acc):
            off = g * L
            lg = l_v[pl.ds(off, L)]
            lpg = lp_v[pl.ds(off, L)]
            cndg = cond_v[pl.ds(off, L)]
            for t in range(L):
                l_r = lg[t]
                lp_r = lpg[t]
                c_r = cndg[t]
                okp = (c_r != 0) & (l_r >= c_lo) & (l_r < c_lo + CS)
                okn = (c_r != 0) & (lp_r >= c_lo) & (lp_r < c_lo + CS)
                rowp = jnp.where(okp, slot * CS + l_r - c_lo, ZREDIR)
                rown = jnp.where(okn, slot * CS + lp_r - c_lo, ZREDIR)
                chp = buf[rowp, pl.ds(off, L)]
                chn = buf[rown, pl.ds(off, L)]
                sel = lane == t
                acc = acc + jnp.where(sel, chp, 0.0) \
                    - jnp.where(sel, chn, 0.0)
            return acc

        return lax.fori_loop(0, RPT // L, inner, acc)

    issue(0, 0, sem0)
    acc = jnp.zeros((L,), jnp.float32)
    if NSTG == 1:
        drain(0, sem0)
        acc = process(0, 0, acc)
    else:
        def body(k, acc):
            s0 = 2 * k
            issue(s0 + 1, 1, sem1)
            drain(0, sem0)
            acc = process(s0, 0, acc)

            @pl.when(s0 + 2 < NSTG)
            def _():
                issue(s0 + 2, 0, sem0)
            drain(1, sem1)
            acc = process(s0 + 1, 1, acc)
            return acc

        acc = lax.fori_loop(0, NSTG // 2, body, acc)
    acc_v[...] = acc
    pltpu.sync_copy(acc_v, out_hbm.at[wid])


def _tc_block_copy(zt_any, vbuf, sems, i, slot):
    return pltpu.make_async_copy(
        zt_any.at[pl.ds(SPLIT_COL, TC_BC), pl.ds(i * TC_BI, TC_BI)],
        vbuf.at[slot], sems.at[slot])


def _tc_dense_body(zt_any, l_ref, lp_ref, cond_ref, out_ref,
                   vbuf, acc_ref, sems):
    i = pl.program_id(0)
    slot = lax.rem(i, 2)

    @pl.when(i == 0)
    def _():
        acc_ref[...] = jnp.zeros((1, TC_BI), jnp.float32)
        _tc_block_copy(zt_any, vbuf, sems, 0, 0).start()

    @pl.when(i + 1 < TC_NI)
    def _():
        _tc_block_copy(zt_any, vbuf, sems, i + 1, lax.rem(i + 1, 2)).start()

    _tc_block_copy(zt_any, vbuf, sems, i, slot).wait()

    zb = vbuf[slot]
    lb = l_ref[0, 0, :].reshape(1, TC_BI)
    lpb = lp_ref[0, 0, :].reshape(1, TC_BI)
    cb = cond_ref[0, 0, :].reshape(1, TC_BI)
    cids = jax.lax.broadcasted_iota(
        jnp.int32, (TC_BC, TC_BI), 0) + SPLIT_COL
    val = jnp.where(cids == lb, zb, 0.0) - jnp.where(cids == lpb, zb, 0.0)
    acc_ref[...] += jnp.sum(val, axis=0, keepdims=True) * cb

    @pl.when(i == TC_NI - 1)
    def _():
        out_ref[...] = jnp.sum(acc_ref[...]).reshape(1, 1)


_tc_dense = pl.pallas_call(
    _tc_dense_body,
    grid=(TC_NI,),
    in_specs=[
        pl.BlockSpec(memory_space=pl.ANY),
        pl.BlockSpec((1, 1, TC_BI), lambda i: (i, 0, 0)),
        pl.BlockSpec((1, 1, TC_BI), lambda i: (i, 0, 0)),
        pl.BlockSpec((1, 1, TC_BI), lambda i: (i, 0, 0)),
    ],
    out_specs=pl.BlockSpec((1, 1), lambda i: (0, 0)),
    out_shape=jax.ShapeDtypeStruct((1, 1), jnp.float32),
    scratch_shapes=[pltpu.VMEM((2, TC_BC, TC_BI), jnp.float32),
                    pltpu.VMEM((1, TC_BI), jnp.float32),
                    pltpu.SemaphoreType.DMA((2,))],
    compiler_params=pltpu.CompilerParams(
        dimension_semantics=("arbitrary",)),
)


def _tc_combine_body(part_ref, tcs_ref, out_ref):
    out_ref[...] = (jnp.sum(part_ref[...]) + tcs_ref[0, 0]).reshape(1, 1)


_tc_combine = pl.pallas_call(
    _tc_combine_body,
    out_shape=jax.ShapeDtypeStruct((1, 1), jnp.float32),
)


def kernel(z, condition, l, l_prime):
    zt = jnp.swapaxes(z, 0, 1)
    li = l.astype(jnp.int32)
    lpi = l_prime.astype(jnp.int32)
    condi = condition.astype(jnp.int32)
    condf = condition.astype(jnp.float32)
    l3 = li.reshape(TC_NI, 1, TC_BI)
    lp3 = lpi.reshape(TC_NI, 1, TC_BI)
    c3 = condf.reshape(TC_NI, 1, TC_BI)
    sc_part = _sc_stream(zt, li, lpi, condi)
    tc_part = _tc_dense(zt, l3, lp3, c3)
    return _tc_combine(sc_part, tc_part)[0, 0]
